# baseline (speedup 1.0000x reference)
import functools

import jax
import jax.numpy as jnp
from jax import lax
from jax.experimental import pallas as pl
from jax.experimental.pallas import tpu as pltpu
from jax.experimental.pallas import tpu_sc as plsc

B = 16384
D = 64
NG = 5
ROWS = 2 + NG
NC = 2
NS = 16
NW = NC * NS
L = 16
NPW = B // NW
M = 128


def _skipgram_gather(center, context, ng_words, in_table, out_table):
    mesh = plsc.VectorSubcoreMesh(core_axis_name="c", subcore_axis_name="s")

    @functools.partial(
        pl.kernel,
        out_type=jax.ShapeDtypeStruct((B * ROWS, D), jnp.float32),
        mesh=mesh,
        scratch_types=[
            pltpu.VMEM((NPW * ROWS,), jnp.int32),
            pltpu.VMEM((M,), jnp.int32),
            pltpu.VMEM((M, D), jnp.float32),
            pltpu.SemaphoreType.DMA,
        ],
        compiler_params=pltpu.CompilerParams(use_tc_tiling_on_sc=False),
    )
    def k(center_h, context_h, ng_h, in_t, out_t, out_h, src_idx, dst_v, rows, sem):
        wid = lax.axis_index("s") * NC + lax.axis_index("c")
        base = wid * NPW

        pltpu.sync_copy(center_h.at[pl.ds(base, NPW)], src_idx.at[pl.ds(0, NPW)])
        pltpu.sync_copy(context_h.at[pl.ds(base, NPW)], src_idx.at[pl.ds(NPW, NPW)])
        pltpu.sync_copy(ng_h.at[pl.ds(base * NG, NPW * NG)],
                        src_idx.at[pl.ds(2 * NPW, NPW * NG)])

        lane = lax.iota(jnp.int32, L)

        def run_phase(table, src_off, nchunks, dst_fn):
            def chunk(c, carry):
                off = src_off + c * M
                for u in range(M // L):
                    kk = c * M + u * L + lane
                    dst_v[pl.ds(u * L, L)] = dst_fn(kk)
                pltpu.async_copy(table.at[src_idx.at[pl.ds(off, M)]], rows, sem).wait()
                pltpu.async_copy(rows, out_h.at[dst_v], sem).wait()
                return carry
            lax.fori_loop(0, nchunks, chunk, 0)

        run_phase(in_t, 0, NPW // M, lambda kk: (base + kk) * ROWS)
        run_phase(out_t, NPW, NPW // M, lambda kk: (base + kk) * ROWS + 1)
        def dst_c(kk):
            q = (kk * 6554) >> 15
            r = kk - q * NG
            return (base + q) * ROWS + 2 + r
        run_phase(out_t, 2 * NPW, NPW * NG // M, dst_c)

    return k(center, context, ng_words, in_table, out_table)


@jax.jit
def kernel(center, context, in_table, out_table, ng_words):
    out = _skipgram_gather(center, context, ng_words, in_table, out_table)
    return out.reshape(B, ROWS, D)


# speedup vs baseline: 1.0040x; 1.0040x over previous
import functools

import numpy as np
import jax
import jax.numpy as jnp
from jax import lax
from jax.experimental import pallas as pl
from jax.experimental.pallas import tpu as pltpu
from jax.experimental.pallas import tpu_sc as plsc

B = 16384
D = 64
NG = 5
ROWS = 2 + NG
NC = 2
NS = 16
NW = NC * NS
NPW = B // NW
M = 128
NCH = NPW * ROWS // M


def _dst_table() -> np.ndarray:
    dst = np.empty((NW, NCH, M), dtype=np.int32)
    for w in range(NW):
        base = w * NPW
        k = np.arange(NPW)
        a = (base + k) * ROWS
        b = a + 1
        kk = np.arange(NPW * NG)
        c = (base + kk // NG) * ROWS + 2 + kk % NG
        dst[w] = np.concatenate([a, b, c]).reshape(NCH, M)
    return dst


_DST_NP = _dst_table()


def _skipgram_gather(center, context, ng_words, dst_h, in_table, out_table):
    mesh = plsc.VectorSubcoreMesh(core_axis_name="c", subcore_axis_name="s")

    @functools.partial(
        pl.kernel,
        out_type=jax.ShapeDtypeStruct((B * ROWS, D), jnp.float32),
        mesh=mesh,
        scratch_types=[
            pltpu.VMEM((NPW * ROWS,), jnp.int32),
            pltpu.VMEM((NCH, M), jnp.int32),
            pltpu.VMEM((M, D), jnp.float32),
            pltpu.VMEM((M, D), jnp.float32),
            pltpu.SemaphoreType.DMA,
            pltpu.SemaphoreType.DMA,
        ],
        compiler_params=pltpu.CompilerParams(use_tc_tiling_on_sc=False),
    )
    def k(center_h, context_h, ng_h, dst_hbm, in_t, out_t, out_h,
          src_idx, dst_l, rows0, rows1, gsem, ssem):
        wid = lax.axis_index("s") * NC + lax.axis_index("c")
        base = wid * NPW

        pltpu.sync_copy(center_h.at[pl.ds(base, NPW)], src_idx.at[pl.ds(0, NPW)])
        pltpu.sync_copy(context_h.at[pl.ds(base, NPW)], src_idx.at[pl.ds(NPW, NPW)])
        pltpu.sync_copy(ng_h.at[pl.ds(base * NG, NPW * NG)],
                        src_idx.at[pl.ds(2 * NPW, NPW * NG)])
        pltpu.sync_copy(dst_hbm.at[wid], dst_l)

        def run_phase(table, c0, nchunks):
            def gather(c, buf):
                src = table.at[src_idx.at[pl.ds((c0 + c) * M, M)]]
                return pltpu.make_async_copy(src, buf, gsem)

            def scatter(c, buf):
                return pltpu.make_async_copy(buf, out_h.at[dst_l.at[c0 + c]], ssem)

            half = nchunks // 2
            gather(0, rows0).start()

            def body(i, carry):
                a = 2 * i

                gather(a, rows0).wait()
                scatter(a, rows0).start()

                @pl.when(i > 0)
                def _():
                    scatter(a - 1, rows1).wait()

                gather(a + 1, rows1).start()
                gather(a + 1, rows1).wait()
                scatter(a + 1, rows1).start()
                scatter(a, rows0).wait()

                @pl.when(i < half - 1)
                def _():
                    gather(a + 2, rows0).start()

                return carry

            lax.fori_loop(0, half, body, 0)
            scatter(nchunks - 1, rows1).wait()

        run_phase(in_t, 0, NPW // M)
        run_phase(out_t, NPW // M, NPW // M)
        run_phase(out_t, 2 * NPW // M, NPW * NG // M)

    return k(center, context, ng_words, dst_h, in_table, out_table)


@jax.jit
def kernel(center, context, in_table, out_table, ng_words):
    out = _skipgram_gather(center, context, ng_words, jnp.asarray(_DST_NP),
                           in_table, out_table)
    return out.reshape(B, ROWS, D)


# speedup vs baseline: 1.0059x; 1.0019x over previous
import functools

import numpy as np
import jax
import jax.numpy as jnp
from jax import lax
from jax.experimental import pallas as pl
from jax.experimental.pallas import tpu as pltpu
from jax.experimental.pallas import tpu_sc as plsc

VOCAB = 1000000
B = 16384
D = 64
NG = 5
ROWS = 2 + NG
NC = 2
NS = 16
NW = NC * NS
NPW = B // NW
M = 128
NCH = NPW * ROWS // M


def _dst_table() -> np.ndarray:
    dst = np.empty((NW, NCH, M), dtype=np.int32)
    for w in range(NW):
        base = w * NPW
        k = np.arange(NPW)
        a = (base + k) * ROWS
        b = a + 1
        kk = np.arange(NPW * NG)
        c = (base + kk // NG) * ROWS + 2 + kk % NG
        dst[w] = np.concatenate([a, b, c]).reshape(NCH, M)
    return dst


_DST_NP = _dst_table()


def _skipgram_gather(center, context, ng_words, dst_h, in_table, out_table):
    mesh = plsc.VectorSubcoreMesh(core_axis_name="c", subcore_axis_name="s")

    @functools.partial(
        pl.kernel,
        out_type=jax.ShapeDtypeStruct((B * ROWS, D), jnp.float32),
        mesh=mesh,
        scratch_types=[
            pltpu.VMEM((NPW * ROWS,), jnp.int32),
            pltpu.VMEM((NCH, M), jnp.int32),
            pltpu.VMEM((M, D), jnp.float32),
            pltpu.VMEM((M, D), jnp.float32),
            pltpu.SemaphoreType.DMA,
            pltpu.SemaphoreType.DMA,
        ],
        compiler_params=pltpu.CompilerParams(use_tc_tiling_on_sc=False),
    )
    def k(center_h, context_h, ng_h, dst_hbm, in_t, out_t, out_h,
          src_idx, dst_l, rows0, rows1, gsem, ssem):
        wid = lax.axis_index("s") * NC + lax.axis_index("c")
        base = wid * NPW

        pltpu.sync_copy(center_h.at[pl.ds(base, NPW)], src_idx.at[pl.ds(0, NPW)])
        pltpu.sync_copy(context_h.at[pl.ds(base, NPW)], src_idx.at[pl.ds(NPW, NPW)])
        pltpu.sync_copy(ng_h.at[pl.ds(base * NG, NPW * NG)],
                        src_idx.at[pl.ds(2 * NPW, NPW * NG)])
        pltpu.sync_copy(dst_hbm.at[wid], dst_l)

        def run_phase(table, c0, nchunks):
            def gather(c, buf):
                src = table.at[src_idx.at[pl.ds((c0 + c) * M, M)]]
                return pltpu.make_async_copy(src, buf, gsem)

            def scatter(c, buf):
                return pltpu.make_async_copy(buf, out_h.at[dst_l.at[c0 + c]], ssem)

            half = nchunks // 2
            gather(0, rows0).start()

            def body(i, carry):
                a = 2 * i

                gather(a, rows0).wait()
                scatter(a, rows0).start()

                @pl.when(i > 0)
                def _():
                    scatter(a - 1, rows1).wait()

                gather(a + 1, rows1).start()
                gather(a + 1, rows1).wait()
                scatter(a + 1, rows1).start()
                scatter(a, rows0).wait()

                @pl.when(i < half - 1)
                def _():
                    gather(a + 2, rows0).start()

                return carry

            lax.fori_loop(0, half, body, 0)
            scatter(nchunks - 1, rows1).wait()

        run_phase(in_t, 0, NPW // M)
        run_phase(out_t, NPW // M, NPW // M)
        run_phase(out_t, 2 * NPW // M, NPW * NG // M)

    return k(center, context, ng_words, dst_h, in_table, out_table)


def _linearize(table):
    t = lax.optimization_barrier(table.reshape(VOCAB // 2, 2 * D))
    return t.reshape(VOCAB, D)


@jax.jit
def kernel(center, context, in_table, out_table, ng_words):
    out = _skipgram_gather(center, context, ng_words, jnp.asarray(_DST_NP),
                           _linearize(in_table), _linearize(out_table))
    return out.reshape(B, ROWS, D)
